# baseline (speedup 1.0000x reference)
import functools

import jax
import jax.numpy as jnp
from jax import lax
from jax.experimental import pallas as pl
from jax.experimental.pallas import tpu as pltpu
from jax.experimental.pallas import tpu_sc as plsc

K = 8192
E = 256
T = 4096
T_BLK = 512
K_BLK = 1024
NTB = T // T_BLK
NKB = K // K_BLK
NCAND_BLK = 4
NCAND = 8


def _exact_tree_distance(rows, x):
    diff = rows - x
    sq = diff * diff
    p = sq[:, :128] + sq[:, 128:]
    q = p[:, 0:8]
    for m in range(1, 16):
        q = q + p[:, 8 * m:8 * (m + 1)]
    return ((q[:, 0:1] + q[:, 4:5]) + (q[:, 2:3] + q[:, 6:7])) + \
           ((q[:, 1:2] + q[:, 5:6]) + (q[:, 3:4] + q[:, 7:8]))


def _topk_body(x_ref, c_ref, cidx_ref, cval, cidx):
    kb = pl.program_id(1)
    x = x_ref[0]
    c = c_ref[...]
    cn = jnp.sum(c * c, axis=1, keepdims=True)
    s = cn - 2.0 * lax.dot_general(c, x, (((1,), (0,)), ((), ())),
                                   preferred_element_type=jnp.float32)
    rows = lax.broadcasted_iota(jnp.int32, (K_BLK, T_BLK), 0)
    for it in range(NCAND_BLK):
        m = jnp.min(s, axis=0, keepdims=True)
        hit = s == m
        a = jnp.min(jnp.where(hit, rows, K_BLK), axis=0, keepdims=True)
        cval[pl.ds(NCAND_BLK * kb + it, 1), :] = m
        cidx[pl.ds(NCAND_BLK * kb + it, 1), :] = a + kb * K_BLK
        if it != NCAND_BLK - 1:
            s = jnp.where(hit, jnp.inf, s)

    @pl.when(kb == NKB - 1)
    def _():
        vals = cval[...]
        idxs = cidx[...]
        for j in range(NCAND):
            m = jnp.min(vals, axis=0, keepdims=True)
            pick = jnp.min(jnp.where(vals == m, idxs, K), axis=0, keepdims=True)
            cidx_ref[0, pl.ds(j, 1), :] = pick
            if j != NCAND - 1:
                vals = jnp.where((vals == m) & (idxs == pick), jnp.inf, vals)


def _candidates(emb3, codebook):
    return pl.pallas_call(
        _topk_body,
        grid=(NTB, NKB),
        in_specs=[
            pl.BlockSpec((1, E, T_BLK), lambda t, k: (t // 2, 0, t % 2)),
            pl.BlockSpec((K_BLK, E), lambda t, k: (k, 0)),
        ],
        out_specs=pl.BlockSpec((1, NCAND, T_BLK), lambda t, k: (t, 0, 0)),
        out_shape=jax.ShapeDtypeStruct((NTB, NCAND, T_BLK), jnp.int32),
        scratch_shapes=[
            pltpu.VMEM((NCAND_BLK * NKB, T_BLK), jnp.float32),
            pltpu.VMEM((NCAND_BLK * NKB, T_BLK), jnp.int32),
        ],
    )(emb3, codebook)


def _make_gather(n_rows):
    info = plsc.get_sparse_core_info()
    nc, ns = info.num_cores, info.num_subcores
    nw = nc * ns
    b_per_w = n_rows // nw
    chunk = 128
    n_chunks = b_per_w // chunk

    mesh = plsc.VectorSubcoreMesh(core_axis_name="c", subcore_axis_name="s")

    @functools.partial(
        pl.kernel, mesh=mesh,
        out_type=jax.ShapeDtypeStruct((n_rows, E), jnp.float32),
        scratch_types=[
            pltpu.VMEM((n_chunks, chunk), jnp.int32),
            pltpu.VMEM((chunk, E), jnp.float32),
            pltpu.SemaphoreType.DMA,
        ],
    )
    def gather_k(table_hbm, idx_hbm, out_hbm, idx_v, rows_v, sem):
        wid = lax.axis_index("s") * nc + lax.axis_index("c")
        base = wid * b_per_w
        for cnk in range(n_chunks):
            pltpu.sync_copy(idx_hbm.at[pl.ds(base + cnk * chunk, chunk)],
                            idx_v.at[cnk])
            pltpu.async_copy(table_hbm.at[idx_v.at[cnk]], rows_v, sem).wait()
            pltpu.sync_copy(rows_v,
                            out_hbm.at[pl.ds(base + cnk * chunk, chunk)])

    return gather_k


def _rescore_body(x_ref, g_ref, it_ref, outq_ref):
    x = x_ref[0]
    best_d = best_i = best_slot = None
    for j in range(NCAND):
        diff = g_ref[j] - x
        sq = diff * diff
        p = sq[0:128, :] + sq[128:256, :]
        q = p[0:8, :]
        for m in range(1, 16):
            q = q + p[8 * m:8 * (m + 1), :]
        r1 = q + jnp.roll(q, -4, axis=0)
        r2 = r1 + jnp.roll(r1, -2, axis=0)
        r3 = r2 + jnp.roll(r2, -1, axis=0)
        d = r3[0:1, :]
        idx_j = it_ref[0, pl.ds(j, 1), :].reshape(1, T_BLK)
        if j == 0:
            best_d, best_i = d, idx_j
            best_slot = jnp.zeros((1, T_BLK), jnp.int32)
        else:
            better = (d < best_d) | ((d == best_d) & (idx_j < best_i))
            best_d = jnp.where(better, d, best_d)
            best_i = jnp.where(better, idx_j, best_i)
            best_slot = jnp.where(better, j, best_slot)
    acc = g_ref[0]
    for j in range(1, NCAND):
        acc = jnp.where(best_slot == j, g_ref[j], acc)
    outq_ref[0] = acc


def _rescore(emb3, gt, cand):
    return pl.pallas_call(
        _rescore_body,
        grid=(NTB,),
        in_specs=[
            pl.BlockSpec((1, E, T_BLK), lambda t: (t // 2, 0, t % 2)),
            pl.BlockSpec((NCAND, E, T_BLK), lambda t: (0, 0, t)),
            pl.BlockSpec((1, NCAND, T_BLK), lambda t: (t, 0, 0)),
        ],
        out_specs=pl.BlockSpec((1, E, T_BLK), lambda t: (t, 0, 0)),
        out_shape=jax.ShapeDtypeStruct((NTB, E, T_BLK), jnp.float32),
    )(emb3, gt, cand)


def kernel(embeddings, codebook):
    B, e, H, W = embeddings.shape
    emb3 = embeddings.reshape(B, e, H * W)
    cand = _candidates(emb3, codebook)
    idx_jmaj = cand.transpose(1, 0, 2).reshape(NCAND * T)
    grows = _make_gather(NCAND * T)(codebook, idx_jmaj)
    gt = grows.reshape(NCAND, T, e).transpose(0, 2, 1)
    outq = _rescore(emb3, gt, cand)
    cols = outq.transpose(1, 0, 2).reshape(e, B, H * W)
    return cols.transpose(1, 0, 2).reshape(B, e, H, W)


# speedup vs baseline: 1.0849x; 1.0849x over previous
import functools

import jax
import jax.numpy as jnp
from jax import lax
from jax.experimental import pallas as pl
from jax.experimental.pallas import tpu as pltpu
from jax.experimental.pallas import tpu_sc as plsc

K = 8192
E = 256
T = 4096
T_BLK = 512
K_BLK = 1024
NTB = T // T_BLK
NKB = K // K_BLK
NCAND_BLK = 4
NCAND = 8


def _exact_tree_distance(rows, x):
    diff = rows - x
    sq = diff * diff
    p = sq[:, :128] + sq[:, 128:]
    q = p[:, 0:8]
    for m in range(1, 16):
        q = q + p[:, 8 * m:8 * (m + 1)]
    return ((q[:, 0:1] + q[:, 4:5]) + (q[:, 2:3] + q[:, 6:7])) + \
           ((q[:, 1:2] + q[:, 5:6]) + (q[:, 3:4] + q[:, 7:8]))


def _topk_body(x_ref, c_ref, cidx_ref, cval, cidx):
    kb = pl.program_id(1)
    x = x_ref[0]
    c = c_ref[...]
    cn = jnp.sum(c * c, axis=1, keepdims=True)
    s = cn - 2.0 * lax.dot_general(c, x, (((1,), (0,)), ((), ())),
                                   preferred_element_type=jnp.float32)
    rowsf = lax.broadcasted_iota(jnp.int32, (K_BLK, T_BLK), 0).astype(jnp.float32)
    kofs = (kb * K_BLK).astype(jnp.float32)
    vs, ix = [], []
    for it in range(NCAND_BLK):
        m = jnp.min(s, axis=0, keepdims=True)
        hit = s == m
        a = jnp.min(jnp.where(hit, rowsf, jnp.float32(K)),
                    axis=0, keepdims=True)
        vs.append(m)
        ix.append(a + kofs)
        if it != NCAND_BLK - 1:
            s = jnp.where(hit, jnp.inf, s)
    pad = jnp.full((8 - NCAND_BLK, T_BLK), jnp.inf, jnp.float32)
    padi = jnp.full((8 - NCAND_BLK, T_BLK), jnp.float32(2 * K), jnp.float32)
    off = pl.multiple_of(8 * kb, 8)
    cval[pl.ds(off, 8), :] = jnp.concatenate(vs + [pad], axis=0)
    cidx[pl.ds(off, 8), :] = jnp.concatenate(ix + [padi], axis=0)

    @pl.when(kb == NKB - 1)
    def _():
        vals = cval[...]
        idxs = cidx[...]
        outs = []
        for j in range(NCAND):
            m = jnp.min(vals, axis=0, keepdims=True)
            hit = vals == m
            pick = jnp.min(jnp.where(hit, idxs, jnp.float32(2 * K)),
                           axis=0, keepdims=True)
            outs.append(pick)
            if j != NCAND - 1:
                vals = jnp.where(hit & (idxs == pick), jnp.inf, vals)
        cidx_ref[0, :, :] = jnp.concatenate(outs, axis=0).astype(jnp.int32)


def _candidates(emb3, codebook):
    return pl.pallas_call(
        _topk_body,
        grid=(NTB, NKB),
        in_specs=[
            pl.BlockSpec((1, E, T_BLK), lambda t, k: (t // 2, 0, t % 2)),
            pl.BlockSpec((K_BLK, E), lambda t, k: (k, 0)),
        ],
        out_specs=pl.BlockSpec((1, NCAND, T_BLK), lambda t, k: (t, 0, 0)),
        out_shape=jax.ShapeDtypeStruct((NTB, NCAND, T_BLK), jnp.int32),
        scratch_shapes=[
            pltpu.VMEM((8 * NKB, T_BLK), jnp.float32),
            pltpu.VMEM((8 * NKB, T_BLK), jnp.float32),
        ],
    )(emb3, codebook)


def _make_gather(n_rows):
    info = plsc.get_sparse_core_info()
    nc, ns = info.num_cores, info.num_subcores
    nw = nc * ns
    b_per_w = n_rows // nw
    chunk = 128
    n_chunks = b_per_w // chunk

    mesh = plsc.VectorSubcoreMesh(core_axis_name="c", subcore_axis_name="s")

    @functools.partial(
        pl.kernel, mesh=mesh,
        out_type=jax.ShapeDtypeStruct((n_rows, E), jnp.float32),
        scratch_types=[
            pltpu.VMEM((n_chunks, chunk), jnp.int32),
            pltpu.VMEM((chunk, E), jnp.float32),
            pltpu.SemaphoreType.DMA,
        ],
    )
    def gather_k(table_hbm, idx_hbm, out_hbm, idx_v, rows_v, sem):
        wid = lax.axis_index("s") * nc + lax.axis_index("c")
        base = wid * b_per_w
        for cnk in range(n_chunks):
            pltpu.sync_copy(idx_hbm.at[pl.ds(base + cnk * chunk, chunk)],
                            idx_v.at[cnk])
            pltpu.async_copy(table_hbm.at[idx_v.at[cnk]], rows_v, sem).wait()
            pltpu.sync_copy(rows_v,
                            out_hbm.at[pl.ds(base + cnk * chunk, chunk)])

    return gather_k


def _rescore_body(x_ref, g_ref, it_ref, outq_ref):
    x = x_ref[0]
    best_d = best_i = best_slot = None
    for j in range(NCAND):
        diff = g_ref[j] - x
        sq = diff * diff
        p = sq[0:128, :] + sq[128:256, :]
        q = p[0:8, :]
        for m in range(1, 16):
            q = q + p[8 * m:8 * (m + 1), :]
        r1 = q + jnp.roll(q, -4, axis=0)
        r2 = r1 + jnp.roll(r1, -2, axis=0)
        r3 = r2 + jnp.roll(r2, -1, axis=0)
        d = r3[0:1, :]
        idx_j = it_ref[0, pl.ds(j, 1), :].reshape(1, T_BLK)
        if j == 0:
            best_d, best_i = d, idx_j
            best_slot = jnp.zeros((1, T_BLK), jnp.int32)
        else:
            better = (d < best_d) | ((d == best_d) & (idx_j < best_i))
            best_d = jnp.where(better, d, best_d)
            best_i = jnp.where(better, idx_j, best_i)
            best_slot = jnp.where(better, j, best_slot)
    acc = g_ref[0]
    for j in range(1, NCAND):
        acc = jnp.where(best_slot == j, g_ref[j], acc)
    outq_ref[0] = acc


def _rescore(emb3, gt, cand):
    return pl.pallas_call(
        _rescore_body,
        grid=(NTB,),
        in_specs=[
            pl.BlockSpec((1, E, T_BLK), lambda t: (t // 2, 0, t % 2)),
            pl.BlockSpec((NCAND, E, T_BLK), lambda t: (0, 0, t)),
            pl.BlockSpec((1, NCAND, T_BLK), lambda t: (t, 0, 0)),
        ],
        out_specs=pl.BlockSpec((1, E, T_BLK), lambda t: (t, 0, 0)),
        out_shape=jax.ShapeDtypeStruct((NTB, E, T_BLK), jnp.float32),
    )(emb3, gt, cand)


def kernel(embeddings, codebook):
    B, e, H, W = embeddings.shape
    emb3 = embeddings.reshape(B, e, H * W)
    cand = _candidates(emb3, codebook)
    idx_jmaj = cand.transpose(1, 0, 2).reshape(NCAND * T)
    grows = _make_gather(NCAND * T)(codebook, idx_jmaj)
    gt = grows.reshape(NCAND, T, e).transpose(0, 2, 1)
    outq = _rescore(emb3, gt, cand)
    cols = outq.transpose(1, 0, 2).reshape(e, B, H * W)
    return cols.transpose(1, 0, 2).reshape(B, e, H, W)


# speedup vs baseline: 1.4615x; 1.3471x over previous
import functools

import jax
import jax.numpy as jnp
from jax import lax
from jax.experimental import pallas as pl
from jax.experimental.pallas import tpu as pltpu
from jax.experimental.pallas import tpu_sc as plsc

K = 8192
E = 256
T = 4096
T_BLK = 512
K_BLK = 2048
NTB = T // T_BLK
NKB = K // K_BLK
NCAND_BLK = 4
NCAND = 8


def _topk_body(x_ref, c_ref, cidx_ref, cval):
    kb = pl.program_id(1)
    x = x_ref[0]
    c = c_ref[...]
    cn = jnp.sum(c * c, axis=1, keepdims=True)
    s = cn - 2.0 * lax.dot_general(c, x, (((1,), (0,)), ((), ())),
                                   preferred_element_type=jnp.float32)
    rows = lax.broadcasted_iota(jnp.int32, (K_BLK, T_BLK), 0)
    inv = (8191 - kb * K_BLK) - rows
    sbits = lax.bitcast_convert_type(s, jnp.int32)
    packed = lax.bitcast_convert_type((sbits & (-8192)) | inv, jnp.float32)
    vs = []
    for it in range(NCAND_BLK):
        m = jnp.min(packed, axis=0, keepdims=True)
        vs.append(m)
        if it != NCAND_BLK - 1:
            packed = jnp.where(packed == m, jnp.inf, packed)
    pad = jnp.full((8 - NCAND_BLK, T_BLK), jnp.inf, jnp.float32)
    off = pl.multiple_of(8 * kb, 8)
    cval[pl.ds(off, 8), :] = jnp.concatenate(vs + [pad], axis=0)

    @pl.when(kb == NKB - 1)
    def _():
        vals = cval[...]
        outs = []
        for j in range(NCAND):
            m = jnp.min(vals, axis=0, keepdims=True)
            outs.append(m)
            if j != NCAND - 1:
                vals = jnp.where(vals == m, jnp.inf, vals)
        allm = jnp.concatenate(outs, axis=0)
        mbits = lax.bitcast_convert_type(allm, jnp.int32)
        cidx_ref[0, :, :] = 8191 - (mbits & 8191)


def _candidates(emb3, codebook):
    return pl.pallas_call(
        _topk_body,
        grid=(NTB, NKB),
        in_specs=[
            pl.BlockSpec((1, E, T_BLK), lambda t, k: (t // 2, 0, t % 2)),
            pl.BlockSpec((K_BLK, E), lambda t, k: (k, 0)),
        ],
        out_specs=pl.BlockSpec((1, NCAND, T_BLK), lambda t, k: (t, 0, 0)),
        out_shape=jax.ShapeDtypeStruct((NTB, NCAND, T_BLK), jnp.int32),
        scratch_shapes=[
            pltpu.VMEM((8 * NKB, T_BLK), jnp.float32),
        ],
    )(emb3, codebook)


def _make_gather(n_rows):
    info = plsc.get_sparse_core_info()
    nc, ns = info.num_cores, info.num_subcores
    nw = nc * ns
    b_per_w = n_rows // nw
    chunk = 128
    n_chunks = b_per_w // chunk

    mesh = plsc.VectorSubcoreMesh(core_axis_name="c", subcore_axis_name="s")

    @functools.partial(
        pl.kernel, mesh=mesh,
        out_type=jax.ShapeDtypeStruct((n_rows, E), jnp.float32),
        scratch_types=[
            pltpu.VMEM((n_chunks, chunk), jnp.int32),
            pltpu.VMEM((chunk, E), jnp.float32),
            pltpu.SemaphoreType.DMA,
        ],
    )
    def gather_k(table_hbm, idx_hbm, out_hbm, idx_v, rows_v, sem):
        wid = lax.axis_index("s") * nc + lax.axis_index("c")
        base = wid * b_per_w
        for cnk in range(n_chunks):
            pltpu.sync_copy(idx_hbm.at[pl.ds(base + cnk * chunk, chunk)],
                            idx_v.at[cnk])
            pltpu.async_copy(table_hbm.at[idx_v.at[cnk]], rows_v, sem).wait()
            pltpu.sync_copy(rows_v,
                            out_hbm.at[pl.ds(base + cnk * chunk, chunk)])

    return gather_k


def _rescore_body(x_ref, g_ref, it_ref, outq_ref):
    x = x_ref[0]
    best_d = best_i = best_slot = None
    for j in range(NCAND):
        diff = g_ref[j] - x
        sq = diff * diff
        p = sq[0:128, :] + sq[128:256, :]
        q = p[0:8, :]
        for m in range(1, 16):
            q = q + p[8 * m:8 * (m + 1), :]
        r1 = q + jnp.roll(q, -4, axis=0)
        r2 = r1 + jnp.roll(r1, -2, axis=0)
        r3 = r2 + jnp.roll(r2, -1, axis=0)
        d = r3[0:1, :]
        idx_j = it_ref[0, pl.ds(j, 1), :].reshape(1, T_BLK)
        if j == 0:
            best_d, best_i = d, idx_j
            best_slot = jnp.zeros((1, T_BLK), jnp.int32)
        else:
            better = (d < best_d) | ((d == best_d) & (idx_j < best_i))
            best_d = jnp.where(better, d, best_d)
            best_i = jnp.where(better, idx_j, best_i)
            best_slot = jnp.where(better, j, best_slot)
    acc = g_ref[0]
    for j in range(1, NCAND):
        acc = jnp.where(best_slot == j, g_ref[j], acc)
    outq_ref[0] = acc


def _rescore(emb3, gt, cand):
    return pl.pallas_call(
        _rescore_body,
        grid=(NTB,),
        in_specs=[
            pl.BlockSpec((1, E, T_BLK), lambda t: (t // 2, 0, t % 2)),
            pl.BlockSpec((NCAND, E, T_BLK), lambda t: (0, 0, t)),
            pl.BlockSpec((1, NCAND, T_BLK), lambda t: (t, 0, 0)),
        ],
        out_specs=pl.BlockSpec((1, E, T_BLK), lambda t: (t, 0, 0)),
        out_shape=jax.ShapeDtypeStruct((NTB, E, T_BLK), jnp.float32),
    )(emb3, gt, cand)


def kernel(embeddings, codebook):
    B, e, H, W = embeddings.shape
    emb3 = embeddings.reshape(B, e, H * W)
    cand = _candidates(emb3, codebook)
    idx_jmaj = cand.transpose(1, 0, 2).reshape(NCAND * T)
    grows = _make_gather(NCAND * T)(codebook, idx_jmaj)
    gt = grows.reshape(NCAND, T, e).transpose(0, 2, 1)
    outq = _rescore(emb3, gt, cand)
    cols = outq.transpose(1, 0, 2).reshape(e, B, H * W)
    return cols.transpose(1, 0, 2).reshape(B, e, H, W)
